# Initial kernel scaffold; baseline (speedup 1.0000x reference)
import jax
import jax.numpy as jnp
from jax.experimental import pallas as pl


def kernel(x, edge_index, batch, W1, b1, W2, b2, W3, b3, W4, b4, Wf, bf):
    raise NotImplementedError("write your pallas kernel here")


# speedup vs baseline: 7.4419x; 7.4419x over previous
import functools

import jax
import jax.numpy as jnp
from jax import lax
from jax.experimental import pallas as pl
from jax.experimental.pallas import tpu as pltpu, tpu_sc as plsc

N = 10000
NPAD = 10240
E = 320000
D = 128
G = 64
NC, NS = 2, 16
NW = NC * NS
C = 80
K = E // (NW * C)
BR = 2000
GRID = N // BR

_mesh = plsc.VectorSubcoreMesh(core_axis_name="c", subcore_axis_name="s")


KG = 25
NG = K // KG


@functools.partial(
    pl.kernel,
    out_type=jax.ShapeDtypeStruct((NC, NPAD, D), jnp.float32),
    mesh=_mesh,
    scratch_types=[
        pltpu.VMEM((KG, C), jnp.int32),
        pltpu.VMEM((KG, C), jnp.int32),
        pltpu.VMEM((C, D), jnp.float32),
        pltpu.VMEM_SHARED((NPAD, D), jnp.float32),
        pltpu.SemaphoreType.DMA,
    ],
)
def _sc_agg(table_hbm, src_hbm, dst_hbm, zeros_hbm, out_hbm,
            src_v, dst_v, rows_v, acc_sh, sem):
    cid = lax.axis_index("c")
    sid = lax.axis_index("s")
    wid = cid * NS + sid

    pltpu.sync_copy(zeros_hbm, rows_v)
    for k in range(8):
        pltpu.sync_copy(rows_v, acc_sh.at[pl.ds(sid * 640 + k * C, C)])
    plsc.subcore_barrier()

    def group(g, carry):
        pltpu.sync_copy(src_hbm.at[wid].at[g], src_v)
        pltpu.sync_copy(dst_hbm.at[wid].at[g], dst_v)

        def body(j, carry2):
            pltpu.async_copy(table_hbm.at[src_v.at[j]], rows_v, sem).wait()
            pltpu.sync_copy(rows_v, acc_sh.at[dst_v.at[j]], add=True)
            return carry2

        return lax.fori_loop(0, KG, body, carry)

    lax.fori_loop(0, NG, group, 0)
    plsc.subcore_barrier()

    for k in range(8):
        off = sid * 640 + k * C
        pltpu.sync_copy(acc_sh.at[pl.ds(off, C)], rows_v)
        pltpu.sync_copy(rows_v, out_hbm.at[cid].at[pl.ds(off, C)])


def _mlp1_body(x_ref, p0_ref, p1_ref, w1_ref, b1_ref, w2_ref, b2_ref, o_ref):
    h = x_ref[...] + p0_ref[0] + p1_ref[0]
    t = jnp.dot(h, w1_ref[...], preferred_element_type=jnp.float32) + b1_ref[...]
    t = jnp.maximum(t, 0.0)
    o = jnp.dot(t, w2_ref[...], preferred_element_type=jnp.float32) + b2_ref[...]
    o_ref[...] = jnp.maximum(o, 0.0)


def _mlp2_body(h_ref, p0_ref, p1_ref, w3_ref, b3_ref, w4_ref, b4_ref,
               batch_ref, wf_ref, bf_ref, o_ref, acc):
    i = pl.program_id(0)
    h = h_ref[...] + p0_ref[0] + p1_ref[0]
    t = jnp.dot(h, w3_ref[...], preferred_element_type=jnp.float32) + b3_ref[...]
    t = jnp.maximum(t, 0.0)
    u = jnp.dot(t, w4_ref[...], preferred_element_type=jnp.float32) + b4_ref[...]
    u = jnp.maximum(u, 0.0)
    onehot = jnp.where(
        batch_ref[0] == lax.broadcasted_iota(jnp.int32, (G, BR), 0), 1.0, 0.0)
    contrib = jnp.dot(onehot, u, preferred_element_type=jnp.float32)

    @pl.when(i == 0)
    def _():
        acc[...] = contrib

    @pl.when(i > 0)
    def _():
        acc[...] += contrib

    @pl.when(i == GRID - 1)
    def _():
        o = jnp.dot(acc[...], wf_ref[...], preferred_element_type=jnp.float32)
        o_ref[...] = jnp.maximum(o + bf_ref[...], 0.0)


_row_spec = pl.BlockSpec((BR, D), lambda i: (i, 0))
_part0_spec = pl.BlockSpec((1, BR, D), lambda i: (0, i, 0))
_part1_spec = pl.BlockSpec((1, BR, D), lambda i: (1, i, 0))
_w_spec = pl.BlockSpec((D, D), lambda i: (0, 0))
_b_spec = pl.BlockSpec((1, D), lambda i: (0, 0))

_mlp1 = pl.pallas_call(
    _mlp1_body,
    grid=(GRID,),
    in_specs=[_row_spec, _part0_spec, _part1_spec,
              _w_spec, _b_spec, _w_spec, _b_spec],
    out_specs=_row_spec,
    out_shape=jax.ShapeDtypeStruct((N, D), jnp.float32),
)

_mlp2 = pl.pallas_call(
    _mlp2_body,
    grid=(GRID,),
    in_specs=[_row_spec, _part0_spec, _part1_spec,
              _w_spec, _b_spec, _w_spec, _b_spec,
              pl.BlockSpec((1, 1, BR), lambda i: (i, 0, 0)),
              _w_spec, _b_spec],
    out_specs=pl.BlockSpec((G, D), lambda i: (0, 0)),
    out_shape=jax.ShapeDtypeStruct((G, D), jnp.float32),
    scratch_shapes=[pltpu.VMEM((G, D), jnp.float32)],
)


def kernel(x, edge_index, batch, W1, b1, W2, b2, W3, b3, W4, b4, Wf, bf):
    src = edge_index[0].reshape(NW, NG, KG, C)
    dst = edge_index[1].reshape(NW, NG, KG, C)
    zeros = jnp.zeros((C, D), jnp.float32)
    b1r, b2r, b3r, b4r, bfr = (b.reshape(1, D) for b in (b1, b2, b3, b4, bf))
    batch3 = batch.reshape(GRID, 1, BR)

    parts1 = _sc_agg(x, src, dst, zeros)
    h1 = _mlp1(x, parts1, parts1, W1.T, b1r, W2.T, b2r)
    parts2 = _sc_agg(h1, src, dst, zeros)
    out = _mlp2(h1, parts2, parts2, W3.T, b3r, W4.T, b4r, batch3, Wf.T, bfr)
    return out
